# Initial kernel scaffold; baseline (speedup 1.0000x reference)
import functools

import jax
import jax.numpy as jnp
from jax import lax
from jax.experimental import pallas as pl
from jax.experimental.pallas import tpu as pltpu
from jax.experimental.pallas import tpu_sc as plsc

N_NODES = 100000
NUM_IN = 128
NUM_OUT = 64
N_EDGES = 1600000
BATCH = 16

L = 16
NS = 16
EPT = N_EDGES // NS
CHUNK = 4000
NCHUNK = EPT // CHUNK
NGROUP = CHUNK // L
CAP = 2048
SPILL = 102400
TRASH = N_NODES
AGG_ROWS = N_NODES + 8
OUT_ROWS = 80
OUT_TRASH = NUM_OUT


def _tanh16(v):
    a = jnp.abs(v)
    t = jnp.exp(a * -2.0)
    r = (1.0 - t) / (1.0 + t)
    return jnp.where(v < 0.0, -r, r)


def _body(xT_hbm, src_hbm, dst_hbm, w_hbm,
          out_hbm, sp_as, sp_ad, sp_aw, sp_bs, sp_bd, sp_bw,
          sbuf, dbuf, wbuf,
          stA_s, stA_d, stA_w, stB_s, stB_d, stB_w,
          zmsg, msg, didx, rows, bidx, outbuf, oidx, obuf, xTv,
          agg_sh, out_sh):
    wid = lax.axis_index("s")

    pltpu.sync_copy(xT_hbm, xTv)
    zero16 = jnp.zeros((L,), jnp.float32)

    def _zrow(r, _):
        zmsg[r] = zero16
        return 0
    lax.fori_loop(0, 128, _zrow, 0)

    def _orow(r, _):
        outbuf[r] = zero16
        return 0
    lax.fori_loop(0, OUT_ROWS, _orow, 0)

    lanes = lax.iota(jnp.int32, L)
    for k in range(OUT_ROWS // L):
        oidx[pl.ds(k * L, L)] = lanes + (k * L)

    @pl.when(wid == 0)
    def _():
        pltpu.sync_copy(zmsg.at[pl.ds(0, OUT_ROWS)], out_sh)

    def _flush(st_s, st_d, st_w, sp_s, sp_d, sp_w, off, nf):
        hb = wid * SPILL + nf * CAP
        pltpu.sync_copy(st_s.at[pl.ds(0, CAP)], sp_s.at[pl.ds(hb, CAP)])
        pltpu.sync_copy(st_d.at[pl.ds(0, CAP)], sp_d.at[pl.ds(hb, CAP)])
        pltpu.sync_copy(st_w.at[pl.ds(0, CAP)], sp_w.at[pl.ds(hb, CAP)])
        ts = st_s[pl.ds(CAP, L)]
        td = st_d[pl.ds(CAP, L)]
        tw = st_w[pl.ds(CAP, L)]
        st_s[pl.ds(0, L)] = ts
        st_d[pl.ds(0, L)] = td
        st_w[pl.ds(0, L)] = tw
        return off - CAP, nf + 1

    def _chunk(it, carry):
        offA, offB, nfA, nfB = carry
        base = wid * EPT + it * CHUNK
        pltpu.sync_copy(src_hbm.at[pl.ds(base, CHUNK)], sbuf)
        pltpu.sync_copy(dst_hbm.at[pl.ds(base, CHUNK)], dbuf)
        pltpu.sync_copy(w_hbm.at[pl.ds(base, CHUNK)], wbuf)

        def _group(g, c):
            offA, offB, nfA, nfB = c
            s = sbuf[pl.ds(g * L, L)]
            d = dbuf[pl.ds(g * L, L)]
            m1 = s < NUM_IN
            m2 = d >= (N_NODES - NUM_OUT)
            c1 = plsc.all_reduce_population_count(m1)[0]
            c2 = plsc.all_reduce_population_count(m2)[0]

            def _storeA(off):
                w = wbuf[pl.ds(g * L, L)]
                plsc.store_compressed(stA_s.at[pl.ds(off, L)], s, mask=m1)
                plsc.store_compressed(stA_d.at[pl.ds(off, L)], d, mask=m1)
                plsc.store_compressed(stA_w.at[pl.ds(off, L)], w, mask=m1)
                return off + c1
            offA = lax.cond(c1 > 0, _storeA, lambda off: off, offA)

            def _storeB(off):
                w = wbuf[pl.ds(g * L, L)]
                plsc.store_compressed(stB_s.at[pl.ds(off, L)], s, mask=m2)
                plsc.store_compressed(stB_d.at[pl.ds(off, L)], d, mask=m2)
                plsc.store_compressed(stB_w.at[pl.ds(off, L)], w, mask=m2)
                return off + c2
            offB = lax.cond(c2 > 0, _storeB, lambda off: off, offB)

            offA, nfA = lax.cond(
                offA >= CAP,
                lambda o, n: _flush(stA_s, stA_d, stA_w, sp_as, sp_ad, sp_aw, o, n),
                lambda o, n: (o, n), offA, nfA)
            offB, nfB = lax.cond(
                offB >= CAP,
                lambda o, n: _flush(stB_s, stB_d, stB_w, sp_bs, sp_bd, sp_bw, o, n),
                lambda o, n: (o, n), offB, nfB)
            return offA, offB, nfA, nfB

        return lax.fori_loop(0, NGROUP, _group, (offA, offB, nfA, nfB))

    offA, offB, nfA, nfB = lax.fori_loop(
        0, NCHUNK, _chunk,
        (jnp.int32(0), jnp.int32(0), jnp.int32(0), jnp.int32(0)))

    def _load_spill(sp_s, sp_d, sp_w, f):
        hb = wid * SPILL + f * CAP
        pltpu.sync_copy(sp_s.at[pl.ds(hb, CAP)], sbuf.at[pl.ds(0, CAP)])
        pltpu.sync_copy(sp_d.at[pl.ds(hb, CAP)], dbuf.at[pl.ds(0, CAP)])
        pltpu.sync_copy(sp_w.at[pl.ds(hb, CAP)], wbuf.at[pl.ds(0, CAP)])

    def _zero_rows(b_s, cnt):
        nch = (cnt + 127) // 128

        def _ch(ci, _):
            lo = ci * 128

            def _fill(e2, _):
                idx = lo + e2
                valid = idx < cnt
                idx_s = jnp.where(valid, idx, 0)
                s_e = b_s[idx_s]
                s_c = jnp.minimum(jnp.maximum(s_e, 0), N_NODES - 1)
                didx[e2] = jnp.where(valid, s_c, TRASH)
                return 0
            lax.fori_loop(0, 128, _fill, 0)
            pltpu.sync_copy(zmsg, agg_sh.at[didx])
            return 0
        lax.fori_loop(0, nch, _ch, 0)

    def _zb(f, _):
        _load_spill(sp_bs, sp_bd, sp_bw, f)
        _zero_rows(sbuf, jnp.int32(CAP))
        return 0
    lax.fori_loop(0, nfB, _zb, 0)
    _zero_rows(stB_s, offB)

    plsc.subcore_barrier()

    def _add_msgs(b_s, b_d, b_w, cnt):
        nch = (cnt + 127) // 128

        def _ch(ci, _):
            lo = ci * 128

            def _fill(e2, _):
                idx = lo + e2
                valid = idx < cnt
                idx_s = jnp.where(valid, idx, 0)
                s_e = b_s[idx_s]
                d_e = b_d[idx_s]
                w_e = jnp.where(valid, b_w[idx_s], 0.0)
                s_c = jnp.minimum(jnp.maximum(s_e, 0), NUM_IN - 1)
                msg[e2] = xTv[s_c] * w_e
                didx[e2] = jnp.where(valid, d_e, TRASH)
                return 0
            lax.fori_loop(0, 128, _fill, 0)
            pltpu.sync_copy(msg, agg_sh.at[didx], add=True)
            return 0
        lax.fori_loop(0, nch, _ch, 0)

    def _ab(f, _):
        _load_spill(sp_as, sp_ad, sp_aw, f)
        _add_msgs(sbuf, dbuf, wbuf, jnp.int32(CAP))
        return 0
    lax.fori_loop(0, nfA, _ab, 0)
    _add_msgs(stA_s, stA_d, stA_w, offA)

    plsc.subcore_barrier()

    def _gather_out(b_s, b_d, b_w, cnt):
        ngr = (cnt + L - 1) // L

        def _gr(g, _):
            lo = g * L
            s = b_s[pl.ds(lo, L)]
            s_c = jnp.minimum(jnp.maximum(s, 0), N_NODES - 1)
            bidx[pl.ds(0, L)] = s_c
            pltpu.sync_copy(agg_sh.at[bidx], rows)

            def _edge(e, _):
                idx = lo + e
                valid = idx < cnt
                idx_s = jnp.where(valid, idx, 0)
                s_e = b_s[idx_s]
                d_e = b_d[idx_s]
                w_e = jnp.where(valid, b_w[idx_s], 0.0)
                o_e = jnp.minimum(jnp.maximum(d_e - (N_NODES - NUM_OUT), 0),
                                  OUT_TRASH)
                o_e = jnp.where(valid, o_e, OUT_TRASH)
                arow = _tanh16(rows[e])
                s_cl = jnp.minimum(jnp.maximum(s_e, 0), NUM_IN - 1)
                xrow = xTv[s_cl]
                use_x = jnp.full((L,), s_e < NUM_IN)
                r1 = jnp.where(use_x, xrow, arow)
                outbuf[o_e] = outbuf[o_e] + r1 * w_e
                return 0
            lax.fori_loop(0, L, _edge, 0)
            return 0
        lax.fori_loop(0, ngr, _gr, 0)

    def _gb(f, _):
        _load_spill(sp_bs, sp_bd, sp_bw, f)
        _gather_out(sbuf, dbuf, wbuf, jnp.int32(CAP))
        return 0
    lax.fori_loop(0, nfB, _gb, 0)
    _gather_out(stB_s, stB_d, stB_w, offB)

    pltpu.sync_copy(outbuf, out_sh.at[oidx], add=True)
    plsc.subcore_barrier()

    pltpu.sync_copy(out_sh.at[pl.ds(wid * 4, 4)], rows.at[pl.ds(0, 4)])
    for r in range(4):
        obuf[r] = _tanh16(rows[r])
    pltpu.sync_copy(obuf, out_hbm.at[pl.ds(wid * 4, 4)])


def _sc_forward(xT_flat, src, dst, w):
    f32 = jnp.float32
    i32 = jnp.int32
    mesh = plsc.VectorSubcoreMesh(
        core_axis_name="c", subcore_axis_name="s", num_cores=1,
        num_subcores=NS)
    out_types = (
        jax.ShapeDtypeStruct((NUM_OUT, BATCH), f32),
        jax.ShapeDtypeStruct((NS * SPILL,), i32),
        jax.ShapeDtypeStruct((NS * SPILL,), i32),
        jax.ShapeDtypeStruct((NS * SPILL,), f32),
        jax.ShapeDtypeStruct((NS * SPILL,), i32),
        jax.ShapeDtypeStruct((NS * SPILL,), i32),
        jax.ShapeDtypeStruct((NS * SPILL,), f32),
    )
    scratch = [
        pltpu.VMEM((CHUNK,), i32),
        pltpu.VMEM((CHUNK,), i32),
        pltpu.VMEM((CHUNK,), f32),
        pltpu.VMEM((CAP + L,), i32),
        pltpu.VMEM((CAP + L,), i32),
        pltpu.VMEM((CAP + L,), f32),
        pltpu.VMEM((CAP + L,), i32),
        pltpu.VMEM((CAP + L,), i32),
        pltpu.VMEM((CAP + L,), f32),
        pltpu.VMEM((128, L), f32),
        pltpu.VMEM((128, L), f32),
        pltpu.VMEM((128,), i32),
        pltpu.VMEM((L, L), f32),
        pltpu.VMEM((L,), i32),
        pltpu.VMEM((OUT_ROWS, L), f32),
        pltpu.VMEM((OUT_ROWS,), i32),
        pltpu.VMEM((4, L), f32),
        pltpu.VMEM((NUM_IN, L), f32),
        pltpu.VMEM_SHARED((AGG_ROWS, L), f32),
        pltpu.VMEM_SHARED((OUT_ROWS, L), f32),
    ]
    fn = pl.kernel(
        _body,
        out_type=out_types,
        mesh=mesh,
        scratch_types=scratch,
    )
    res = fn(xT_flat, src, dst, w)
    return res[0]


def kernel(x, edge_index, weights, steps):
    del steps
    xT = jnp.transpose(x)
    src = edge_index[0]
    dst = edge_index[1]
    out_t = _sc_forward(xT.reshape(-1), src, dst, weights)
    return jnp.transpose(out_t)


# speedup vs baseline: 36.4909x; 36.4909x over previous
import jax
import jax.numpy as jnp
from jax import lax
from jax.experimental import pallas as pl
from jax.experimental.pallas import tpu as pltpu
from jax.experimental.pallas import tpu_sc as plsc

N_NODES = 100000
NUM_IN = 128
NUM_OUT = 64
N_EDGES = 1600000
BATCH = 16

L = 16
NC = 2
NS = 16
NT = NC * NS
EPT = N_EDGES // NT
CHUNK = 2000
NCHUNK = EPT // CHUNK
NGROUP = CHUNK // L
CAP = 2048
NBLK = 25
SPILL = NBLK * CAP
HALF = N_NODES // 2
AGG_ROWS = HALF + L
TRASH = HALF
OUT_ROWS = 80
OUT_TRASH = NUM_OUT
CPAD = 16

_params = pltpu.CompilerParams(needs_layout_passes=False)


def _mesh():
    return plsc.VectorSubcoreMesh(
        core_axis_name="c", subcore_axis_name="s", num_cores=NC,
        num_subcores=NS)


def _tanh16(v):
    a = jnp.abs(v)
    t = jnp.exp(a * -2.0)
    r = (1.0 - t) / (1.0 + t)
    return jnp.where(v < 0.0, -r, r)


def _k1_body(src_hbm, dst_hbm, w_hbm,
             sp_as, sp_ad, sp_aw, sp_bs, sp_bd, sp_bw, cnt_hbm,
             sbuf, dbuf, wbuf,
             stA_s, stA_d, stA_w, stB_s, stB_d, stB_w, cbuf):
    cid = lax.axis_index("c")
    sid = lax.axis_index("s")
    tid = cid * NS + sid
    lanes = lax.iota(jnp.int32, L)

    def _flush(st_s, st_d, st_w, sp_s, sp_d, sp_w, off, nf):
        hb = tid * SPILL + nf * CAP
        pltpu.sync_copy(st_s.at[pl.ds(0, CAP)], sp_s.at[pl.ds(hb, CAP)])
        pltpu.sync_copy(st_d.at[pl.ds(0, CAP)], sp_d.at[pl.ds(hb, CAP)])
        pltpu.sync_copy(st_w.at[pl.ds(0, CAP)], sp_w.at[pl.ds(hb, CAP)])
        ts = st_s[pl.ds(CAP, L)]
        td = st_d[pl.ds(CAP, L)]
        tw = st_w[pl.ds(CAP, L)]
        st_s[pl.ds(0, L)] = ts
        st_d[pl.ds(0, L)] = td
        st_w[pl.ds(0, L)] = tw
        return off - CAP, nf + 1

    def _chunk(it, carry):
        offA, offB, nfA, nfB = carry
        base = tid * EPT + it * CHUNK
        pltpu.sync_copy(src_hbm.at[pl.ds(base, CHUNK)], sbuf)
        pltpu.sync_copy(dst_hbm.at[pl.ds(base, CHUNK)], dbuf)
        pltpu.sync_copy(w_hbm.at[pl.ds(base, CHUNK)], wbuf)

        def _group(g, c):
            offA, offB, nfA, nfB = c
            s = sbuf[pl.ds(g * L, L)]
            d = dbuf[pl.ds(g * L, L)]
            m1 = s < NUM_IN
            m2 = d >= (N_NODES - NUM_OUT)
            c1 = jnp.sum(jnp.where(m1, 1, 0).astype(jnp.int32))
            c2 = jnp.sum(jnp.where(m2, 1, 0).astype(jnp.int32))

            def _storeA(off):
                w = wbuf[pl.ds(g * L, L)]
                plsc.store_compressed(stA_s.at[pl.ds(off, L)], s, mask=m1)
                plsc.store_compressed(stA_d.at[pl.ds(off, L)], d, mask=m1)
                plsc.store_compressed(stA_w.at[pl.ds(off, L)], w, mask=m1)
                return off + c1
            offA = lax.cond(c1 > 0, _storeA, lambda off: off, offA)

            def _storeB(off):
                w = wbuf[pl.ds(g * L, L)]
                plsc.store_compressed(stB_s.at[pl.ds(off, L)], s, mask=m2)
                plsc.store_compressed(stB_d.at[pl.ds(off, L)], d, mask=m2)
                plsc.store_compressed(stB_w.at[pl.ds(off, L)], w, mask=m2)
                return off + c2
            offB = lax.cond(c2 > 0, _storeB, lambda off: off, offB)

            offA, nfA = lax.cond(
                offA >= CAP,
                lambda o, n: _flush(stA_s, stA_d, stA_w, sp_as, sp_ad, sp_aw, o, n),
                lambda o, n: (o, n), offA, nfA)
            offB, nfB = lax.cond(
                offB >= CAP,
                lambda o, n: _flush(stB_s, stB_d, stB_w, sp_bs, sp_bd, sp_bw, o, n),
                lambda o, n: (o, n), offB, nfB)
            return offA, offB, nfA, nfB

        return lax.fori_loop(0, NGROUP, _group, (offA, offB, nfA, nfB))

    offA, offB, nfA, nfB = lax.fori_loop(
        0, NCHUNK, _chunk,
        (jnp.int32(0), jnp.int32(0), jnp.int32(0), jnp.int32(0)))

    hbA = tid * SPILL + nfA * CAP
    pltpu.sync_copy(stA_s.at[pl.ds(0, CAP)], sp_as.at[pl.ds(hbA, CAP)])
    pltpu.sync_copy(stA_d.at[pl.ds(0, CAP)], sp_ad.at[pl.ds(hbA, CAP)])
    pltpu.sync_copy(stA_w.at[pl.ds(0, CAP)], sp_aw.at[pl.ds(hbA, CAP)])
    hbB = tid * SPILL + nfB * CAP
    pltpu.sync_copy(stB_s.at[pl.ds(0, CAP)], sp_bs.at[pl.ds(hbB, CAP)])
    pltpu.sync_copy(stB_d.at[pl.ds(0, CAP)], sp_bd.at[pl.ds(hbB, CAP)])
    pltpu.sync_copy(stB_w.at[pl.ds(0, CAP)], sp_bw.at[pl.ds(hbB, CAP)])

    totalA = nfA * CAP + offA
    totalB = nfB * CAP + offB
    lanes_i = lanes
    cv = jnp.where(lanes_i == 0, totalA, jnp.where(lanes_i == 1, totalB, 0))
    cbuf[pl.ds(0, L)] = cv
    pltpu.sync_copy(cbuf, cnt_hbm.at[pl.ds(tid * CPAD, CPAD)])


NBB = 256


def _k2_body(xT_hbm, sp_as, sp_ad, sp_aw, sp_bs, sp_bd, sp_bw, cnt_hbm,
             part_hbm,
             abuf_s, abuf_d, abuf_w, bbuf_s, bbuf_d, bbuf_w,
             accum, outbuf, cntv, xTv):
    cid = lax.axis_index("c")
    sid = lax.axis_index("s")
    g = cid * NS + sid
    lanes = lax.iota(jnp.int32, L)
    zero16 = jnp.zeros((L,), jnp.float32)

    pltpu.sync_copy(xT_hbm, xTv)
    pltpu.sync_copy(cnt_hbm, cntv)
    for r in range(OUT_ROWS):
        outbuf[r] = zero16

    cg = cntv[pl.ds(g * CPAD, L)]
    totB = cg[1]
    nbbat = (totB + NBB - 1) // NBB

    def _bbatch(bb, _):
        boff = bb * NBB
        pltpu.sync_copy(sp_bs.at[pl.ds(g * SPILL + boff, NBB)], bbuf_s)
        pltpu.sync_copy(sp_bd.at[pl.ds(g * SPILL + boff, NBB)], bbuf_d)
        pltpu.sync_copy(sp_bw.at[pl.ds(g * SPILL + boff, NBB)], bbuf_w)
        cntbb = jnp.minimum(totB - boff, NBB)
        ngrB = (cntbb + L - 1) // L

        for r in range(NBB):
            accum[r] = zero16

        def _aseg(t2, _):
            ct = cntv[pl.ds(t2 * CPAD, L)]
            totA = ct[0]
            nblkA = (totA + CAP - 1) // CAP

            def _ablk(ia, _):
                hb = t2 * SPILL + ia * CAP
                pltpu.sync_copy(sp_as.at[pl.ds(hb, CAP)], abuf_s)
                pltpu.sync_copy(sp_ad.at[pl.ds(hb, CAP)], abuf_d)
                pltpu.sync_copy(sp_aw.at[pl.ds(hb, CAP)], abuf_w)
                cnta = jnp.minimum(totA - ia * CAP, CAP)
                ngrA = (cnta + L - 1) // L

                def _agrp(ga, _):
                    lo2 = ga * L
                    validA = (lo2 + lanes) < cnta
                    a_d = abuf_d[pl.ds(lo2, L)]
                    a_s = abuf_s[pl.ds(lo2, L)]
                    a_w = abuf_w[pl.ds(lo2, L)]
                    a_dm = jnp.where(validA, a_d, -2)
                    a_wm = jnp.where(validA, a_w, 0.0)
                    a_sc = jnp.minimum(jnp.maximum(a_s, 0), NUM_IN - 1)

                    def _bgrp(gb, _):
                        lob = gb * L
                        validB = (lob + lanes) < cntbb
                        b_s = bbuf_s[pl.ds(lob, L)]
                        b_sm = jnp.where(validB, b_s, -1)
                        for e in range(L):
                            mk = a_dm == jnp.full((L,), b_sm[e], jnp.int32)
                            c = jnp.sum(jnp.where(mk, 1, 0).astype(jnp.int32))

                            @pl.when(c > 0)
                            def _():
                                row = jnp.full((L,), lob + e, jnp.int32)
                                cur = plsc.load_gather(accum, [row, lanes])
                                add = jnp.zeros((L,), jnp.float32)
                                for b in range(BATCH):
                                    colx = plsc.load_gather(
                                        xTv, [a_sc, jnp.full((L,), b, jnp.int32)])
                                    cb = jnp.sum(jnp.where(mk, a_wm * colx, 0.0))
                                    add = add + jnp.where(lanes == b, cb, 0.0)
                                plsc.store_scatter(accum, [row, lanes], cur + add)
                        return 0
                    lax.fori_loop(0, ngrB, _bgrp, 0)
                    return 0
                lax.fori_loop(0, ngrA, _agrp, 0)
                return 0
            lax.fori_loop(0, nblkA, _ablk, 0)
            return 0
        lax.fori_loop(0, NT, _aseg, 0)

        def _bfin(gb, _):
            lob = gb * L
            validB = (lob + lanes) < cntbb
            b_s = bbuf_s[pl.ds(lob, L)]
            b_d = bbuf_d[pl.ds(lob, L)]
            b_w = bbuf_w[pl.ds(lob, L)]
            w_m = jnp.where(validB, b_w, 0.0)
            o_m = jnp.minimum(jnp.maximum(b_d - (N_NODES - NUM_OUT), 0),
                              OUT_TRASH)
            o_m = jnp.where(validB, o_m, OUT_TRASH)
            s_cx = jnp.minimum(jnp.maximum(b_s, 0), NUM_IN - 1)
            for e in range(L):
                w_e = w_m[e]
                oe = jnp.full((L,), o_m[e], jnp.int32)
                se = jnp.full((L,), s_cx[e], jnp.int32)
                fx = jnp.full((L,), b_s[e], jnp.int32) < NUM_IN
                row = jnp.full((L,), lob + e, jnp.int32)
                av = plsc.load_gather(accum, [row, lanes])
                arow = _tanh16(av)
                xrow = plsc.load_gather(xTv, [se, lanes])
                r1 = jnp.where(fx, xrow, arow)
                cur = plsc.load_gather(outbuf, [oe, lanes])
                plsc.store_scatter(outbuf, [oe, lanes], cur + r1 * w_e)
            return 0
        lax.fori_loop(0, ngrB, _bfin, 0)
        return 0
    lax.fori_loop(0, nbbat, _bbatch, 0)

    pltpu.sync_copy(outbuf.at[pl.ds(0, NUM_OUT)],
                    part_hbm.at[pl.ds(g * NUM_OUT, NUM_OUT)])


def _k3_body(part_hbm, out_hbm, acc, tmp):
    cid = lax.axis_index("c")
    sid = lax.axis_index("s")

    @pl.when(cid == 0)
    def _():
        for r in range(4):
            acc[r] = jnp.zeros((L,), jnp.float32)

        def _t(t, _):
            pltpu.sync_copy(part_hbm.at[pl.ds(t * NUM_OUT + sid * 4, 4)], tmp)
            for r in range(4):
                acc[r] = acc[r] + tmp[r]
            return 0
        lax.fori_loop(0, NT, _t, 0)
        for r in range(4):
            acc[r] = _tanh16(acc[r])
        pltpu.sync_copy(acc, out_hbm.at[pl.ds(sid * 4, 4)])


def _sc_forward(xT, src, dst, w):
    f32 = jnp.float32
    i32 = jnp.int32
    sds = jax.ShapeDtypeStruct

    k1 = pl.kernel(
        _k1_body,
        out_type=(
            sds((NT * SPILL,), i32), sds((NT * SPILL,), i32),
            sds((NT * SPILL,), f32), sds((NT * SPILL,), i32),
            sds((NT * SPILL,), i32), sds((NT * SPILL,), f32),
            sds((NT * CPAD,), i32),
        ),
        mesh=_mesh(),
        scratch_types=[
            pltpu.VMEM((CHUNK,), i32),
            pltpu.VMEM((CHUNK,), i32),
            pltpu.VMEM((CHUNK,), f32),
            pltpu.VMEM((CAP + L,), i32),
            pltpu.VMEM((CAP + L,), i32),
            pltpu.VMEM((CAP + L,), f32),
            pltpu.VMEM((CAP + L,), i32),
            pltpu.VMEM((CAP + L,), i32),
            pltpu.VMEM((CAP + L,), f32),
            pltpu.VMEM((CPAD,), i32),
        ],
        compiler_params=_params,
    )
    sp_as, sp_ad, sp_aw, sp_bs, sp_bd, sp_bw, cnts = k1(src, dst, w)

    k2 = pl.kernel(
        _k2_body,
        out_type=sds((NT * NUM_OUT, BATCH), f32),
        mesh=_mesh(),
        scratch_types=[
            pltpu.VMEM((CAP,), i32),
            pltpu.VMEM((CAP,), i32),
            pltpu.VMEM((CAP,), f32),
            pltpu.VMEM((NBB,), i32),
            pltpu.VMEM((NBB,), i32),
            pltpu.VMEM((NBB,), f32),
            pltpu.VMEM((NBB, L), f32),
            pltpu.VMEM((OUT_ROWS, L), f32),
            pltpu.VMEM((NT * CPAD,), i32),
            pltpu.VMEM((NUM_IN, L), f32),
        ],
        compiler_params=_params,
    )
    part = k2(xT, sp_as, sp_ad, sp_aw, sp_bs, sp_bd, sp_bw, cnts)

    k3 = pl.kernel(
        _k3_body,
        out_type=sds((NUM_OUT, BATCH), f32),
        mesh=_mesh(),
        scratch_types=[
            pltpu.VMEM((4, L), f32),
            pltpu.VMEM((4, L), f32),
        ],
        compiler_params=_params,
    )
    return k3(part)


def kernel(x, edge_index, weights, steps):
    del steps
    xT = jnp.transpose(x)
    src = edge_index[0]
    dst = edge_index[1]
    out_t = _sc_forward(xT, src, dst, weights)
    return jnp.transpose(out_t)
